# baseline (speedup 1.0000x reference)
import functools

import jax
import jax.numpy as jnp
from jax.experimental import pallas as pl
from jax.experimental.pallas import tpu as pltpu

N = 2048
IN_F = 128
OUT_F = 32
AUG = 64
ALPHA = 0.2
BM = 1024
BN = 512
NI = N // BM
NJ = N // BN

_CONTRACT_LAST = (((1,), (1,)), ((), ()))


def _gat_kernel(x_ref, adj_ref, w_ref, a_ref, out_ref,
                haug_ref, s1_ref, s2_ref, acc_ref):
    i = pl.program_id(0)
    j = pl.program_id(1)

    @pl.when((i == 0) & (j == 0))
    def _prologue():
        h = jnp.dot(x_ref[...], w_ref[...],
                    preferred_element_type=jnp.float32,
                    precision=jax.lax.Precision.HIGHEST)
        ones = jnp.ones((N, 1), dtype=jnp.float32)
        zeros = jnp.zeros((N, AUG - OUT_F - 1), dtype=jnp.float32)
        haug_ref[...] = jnp.concatenate([h, ones, zeros], axis=1)
        scale = -1.4426950408889634
        s1_ref[...] = jax.lax.dot_general(
            h, scale * a_ref[0:1, :OUT_F], _CONTRACT_LAST,
            preferred_element_type=jnp.float32,
            precision=jax.lax.Precision.HIGHEST)
        s2_ref[...] = jax.lax.dot_general(
            scale * a_ref[0:1, OUT_F:], h, _CONTRACT_LAST,
            preferred_element_type=jnp.float32,
            precision=jax.lax.Precision.HIGHEST)

    s1b = s1_ref[pl.ds(i * BM, BM), :]
    t = s1b + s2_ref[:, pl.ds(j * BN, BN)]
    arg = jnp.minimum(t, ALPHA * t)
    ee = jnp.where(adj_ref[...] != 0, jnp.exp2(arg), 0.0)
    part = jnp.dot(ee, haug_ref[pl.ds(j * BN, BN), :],
                   preferred_element_type=jnp.float32)

    @pl.when(j == 0)
    def _init():
        acc_ref[...] = part

    @pl.when(j > 0)
    def _accum():
        acc_ref[...] += part

    @pl.when(j == NJ - 1)
    def _epilogue():
        acc = acc_ref[...]
        hp = acc[:, :OUT_F] / acc[:, OUT_F:OUT_F + 1]
        out_ref[...] = jnp.where(hp > 0, hp, jnp.exp(hp) - 1.0)


@functools.partial(jax.jit, static_argnames=())
def kernel(input, adj, W, a):
    return pl.pallas_call(
        _gat_kernel,
        grid=(NI, NJ),
        in_specs=[
            pl.BlockSpec((N, IN_F), lambda i, j: (0, 0)),
            pl.BlockSpec((BM, BN), lambda i, j: (i, j)),
            pl.BlockSpec((IN_F, OUT_F), lambda i, j: (0, 0)),
            pl.BlockSpec((1, 2 * OUT_F), lambda i, j: (0, 0)),
        ],
        out_specs=pl.BlockSpec((BM, OUT_F), lambda i, j: (i, 0)),
        out_shape=jax.ShapeDtypeStruct((N, OUT_F), jnp.float32),
        scratch_shapes=[
            pltpu.VMEM((N, AUG), jnp.float32),
            pltpu.VMEM((N, 1), jnp.float32),
            pltpu.VMEM((1, N), jnp.float32),
            pltpu.VMEM((BM, AUG), jnp.float32),
        ],
        compiler_params=pltpu.CompilerParams(
            dimension_semantics=("arbitrary", "arbitrary"),
        ),
    )(input, adj, W, a)


# speedup vs baseline: 1.1821x; 1.1821x over previous
import functools

import jax
import jax.numpy as jnp
from jax.experimental import pallas as pl
from jax.experimental.pallas import tpu as pltpu

N = 2048
IN_F = 128
OUT_F = 32
AUG = 64
ALPHA = 0.2
CH = 256
NCH = N // CH
NBUF = 3

_CONTRACT_LAST = (((1,), (1,)), ((), ()))


def _gat_kernel(x_ref, adj_hbm, w_ref, a_ref, out_ref,
                haug_ref, s1_ref, s2_ref, bufs, sems):

    def start_copy(k, slot):
        pltpu.make_async_copy(
            adj_hbm.at[pl.ds(k * CH, CH), :], bufs.at[slot], sems.at[slot],
        ).start()

    for k in range(NBUF):
        start_copy(k, k)

    h = jnp.dot(x_ref[...], w_ref[...],
                preferred_element_type=jnp.float32,
                precision=jax.lax.Precision.HIGHEST)
    ones = jnp.ones((N, 1), dtype=jnp.float32)
    zeros = jnp.zeros((N, AUG - OUT_F - 1), dtype=jnp.float32)
    haug_ref[...] = jnp.concatenate([h, ones, zeros], axis=1)
    scale = -1.4426950408889634
    s1_ref[...] = jax.lax.dot_general(
        h, scale * a_ref[0:1, :OUT_F], _CONTRACT_LAST,
        preferred_element_type=jnp.float32,
        precision=jax.lax.Precision.HIGHEST)
    s2_ref[...] = jax.lax.dot_general(
        scale * a_ref[0:1, OUT_F:], h, _CONTRACT_LAST,
        preferred_element_type=jnp.float32,
        precision=jax.lax.Precision.HIGHEST)

    for k in range(NCH):
        slot = k % NBUF
        pltpu.make_async_copy(
            adj_hbm.at[pl.ds(k * CH, CH), :], bufs.at[slot], sems.at[slot],
        ).wait()
        adj_blk = bufs[slot]
        s1b = s1_ref[pl.ds(k * CH, CH), :]
        t = s1b + s2_ref[...]
        arg = jnp.minimum(t, ALPHA * t)
        ee = jnp.where(adj_blk != 0, jnp.exp2(arg), 0.0)
        hp_aug = jnp.dot(ee, haug_ref[...],
                         preferred_element_type=jnp.float32)
        hp = hp_aug[:, :OUT_F] / hp_aug[:, OUT_F:OUT_F + 1]
        out_ref[pl.ds(k * CH, CH), :] = jnp.where(
            hp > 0, hp, jnp.exp(hp) - 1.0)
        if k + NBUF < NCH:
            start_copy(k + NBUF, slot)


@functools.partial(jax.jit, static_argnames=())
def kernel(input, adj, W, a):
    return pl.pallas_call(
        _gat_kernel,
        in_specs=[
            pl.BlockSpec(memory_space=pltpu.VMEM),
            pl.BlockSpec(memory_space=pl.ANY),
            pl.BlockSpec(memory_space=pltpu.VMEM),
            pl.BlockSpec(memory_space=pltpu.VMEM),
        ],
        out_specs=pl.BlockSpec(memory_space=pltpu.VMEM),
        out_shape=jax.ShapeDtypeStruct((N, OUT_F), jnp.float32),
        scratch_shapes=[
            pltpu.VMEM((N, AUG), jnp.float32),
            pltpu.VMEM((N, 1), jnp.float32),
            pltpu.VMEM((1, N), jnp.float32),
            pltpu.VMEM((NBUF, CH, N), jnp.int32),
            pltpu.SemaphoreType.DMA((NBUF,)),
        ],
    )(input, adj, W, a)


# speedup vs baseline: 1.2703x; 1.0746x over previous
import functools

import jax
import jax.numpy as jnp
from jax.experimental import pallas as pl
from jax.experimental.pallas import tpu as pltpu

N = 2048
IN_F = 128
OUT_F = 32
AUG = 64
ALPHA = 0.2
CH = 512
NCH = N // CH
NBUF = 3

_CONTRACT_LAST = (((1,), (1,)), ((), ()))


def _gat_kernel(x_ref, adj_hbm, w_ref, a_ref, out_ref,
                haug_ref, s1_ref, s2_ref, bufs, sems):

    def start_copy(k, slot):
        pltpu.make_async_copy(
            adj_hbm.at[pl.ds(k * CH, CH), :], bufs.at[slot], sems.at[slot],
        ).start()

    for k in range(NBUF):
        start_copy(k, k)

    h = jnp.dot(x_ref[...], w_ref[...],
                preferred_element_type=jnp.float32,
                precision=jax.lax.Precision.HIGHEST)
    ones = jnp.ones((N, 1), dtype=jnp.float32)
    zeros = jnp.zeros((N, AUG - OUT_F - 1), dtype=jnp.float32)
    haug_ref[...] = jnp.concatenate([h, ones, zeros], axis=1)
    scale = -1.4426950408889634
    s1_ref[...] = jax.lax.dot_general(
        h, scale * a_ref[0:1, :OUT_F], _CONTRACT_LAST,
        preferred_element_type=jnp.float32,
        precision=jax.lax.Precision.HIGHEST)
    s2_ref[...] = jax.lax.dot_general(
        scale * a_ref[0:1, OUT_F:], h, _CONTRACT_LAST,
        preferred_element_type=jnp.float32,
        precision=jax.lax.Precision.HIGHEST)

    for k in range(NCH):
        slot = k % NBUF
        pltpu.make_async_copy(
            adj_hbm.at[pl.ds(k * CH, CH), :], bufs.at[slot], sems.at[slot],
        ).wait()
        adj_blk = bufs[slot]
        s1b = s1_ref[pl.ds(k * CH, CH), :]
        t = s1b + s2_ref[...]
        arg = jnp.minimum(t, ALPHA * t)
        ee = jnp.where(adj_blk != 0, jnp.exp2(arg), 0.0)
        hp_aug = jnp.dot(ee, haug_ref[...],
                         preferred_element_type=jnp.float32)
        hp = hp_aug[:, :OUT_F] / hp_aug[:, OUT_F:OUT_F + 1]
        out_ref[pl.ds(k * CH, CH), :] = jnp.where(
            hp > 0, hp, jnp.exp(hp) - 1.0)
        if k + NBUF < NCH:
            start_copy(k + NBUF, slot)


@functools.partial(jax.jit, static_argnames=())
def kernel(input, adj, W, a):
    return pl.pallas_call(
        _gat_kernel,
        in_specs=[
            pl.BlockSpec(memory_space=pltpu.VMEM),
            pl.BlockSpec(memory_space=pl.ANY),
            pl.BlockSpec(memory_space=pltpu.VMEM),
            pl.BlockSpec(memory_space=pltpu.VMEM),
        ],
        out_specs=pl.BlockSpec(memory_space=pltpu.VMEM),
        out_shape=jax.ShapeDtypeStruct((N, OUT_F), jnp.float32),
        scratch_shapes=[
            pltpu.VMEM((N, AUG), jnp.float32),
            pltpu.VMEM((N, 1), jnp.float32),
            pltpu.VMEM((1, N), jnp.float32),
            pltpu.VMEM((NBUF, CH, N), jnp.int32),
            pltpu.SemaphoreType.DMA((NBUF,)),
        ],
    )(input, adj, W, a)


# speedup vs baseline: 1.2850x; 1.0116x over previous
import functools

import jax
import jax.numpy as jnp
from jax.experimental import pallas as pl
from jax.experimental.pallas import tpu as pltpu

N = 2048
IN_F = 128
OUT_F = 32
AUG = 64
ALPHA = 0.2
CH = 1024
NCH = N // CH
NBUF = 2

_CONTRACT_LAST = (((1,), (1,)), ((), ()))


def _gat_kernel(x_ref, adj_hbm, w_ref, a_ref, out_ref,
                haug_ref, s1_ref, s2_ref, bufs, sems):

    def start_copy(k, slot):
        pltpu.make_async_copy(
            adj_hbm.at[pl.ds(k * CH, CH), :], bufs.at[slot], sems.at[slot],
        ).start()

    for k in range(NBUF):
        start_copy(k, k)

    h = jnp.dot(x_ref[...], w_ref[...],
                preferred_element_type=jnp.float32,
                precision=jax.lax.Precision.HIGHEST)
    ones = jnp.ones((N, 1), dtype=jnp.float32)
    zeros = jnp.zeros((N, AUG - OUT_F - 1), dtype=jnp.float32)
    haug_ref[...] = jnp.concatenate([h, ones, zeros], axis=1)
    scale = -1.4426950408889634
    s1_ref[...] = jax.lax.dot_general(
        h, scale * a_ref[0:1, :OUT_F], _CONTRACT_LAST,
        preferred_element_type=jnp.float32,
        precision=jax.lax.Precision.HIGHEST)
    s2_ref[...] = jax.lax.dot_general(
        scale * a_ref[0:1, OUT_F:], h, _CONTRACT_LAST,
        preferred_element_type=jnp.float32,
        precision=jax.lax.Precision.HIGHEST)

    for k in range(NCH):
        slot = k % NBUF
        pltpu.make_async_copy(
            adj_hbm.at[pl.ds(k * CH, CH), :], bufs.at[slot], sems.at[slot],
        ).wait()
        adj_blk = bufs[slot]
        s1b = s1_ref[pl.ds(k * CH, CH), :]
        t = s1b + s2_ref[...]
        arg = jnp.minimum(t, ALPHA * t)
        ee = jnp.where(adj_blk != 0, jnp.exp2(arg), 0.0)
        hp_aug = jnp.dot(ee, haug_ref[...],
                         preferred_element_type=jnp.float32)
        hp = hp_aug[:, :OUT_F] / hp_aug[:, OUT_F:OUT_F + 1]
        out_ref[pl.ds(k * CH, CH), :] = jnp.where(
            hp > 0, hp, jnp.exp(hp) - 1.0)
        if k + NBUF < NCH:
            start_copy(k + NBUF, slot)


@functools.partial(jax.jit, static_argnames=())
def kernel(input, adj, W, a):
    return pl.pallas_call(
        _gat_kernel,
        in_specs=[
            pl.BlockSpec(memory_space=pltpu.VMEM),
            pl.BlockSpec(memory_space=pl.ANY),
            pl.BlockSpec(memory_space=pltpu.VMEM),
            pl.BlockSpec(memory_space=pltpu.VMEM),
        ],
        out_specs=pl.BlockSpec(memory_space=pltpu.VMEM),
        out_shape=jax.ShapeDtypeStruct((N, OUT_F), jnp.float32),
        scratch_shapes=[
            pltpu.VMEM((N, AUG), jnp.float32),
            pltpu.VMEM((N, 1), jnp.float32),
            pltpu.VMEM((1, N), jnp.float32),
            pltpu.VMEM((NBUF, CH, N), jnp.int32),
            pltpu.SemaphoreType.DMA((NBUF,)),
        ],
    )(input, adj, W, a)
